# baseline (speedup 1.0000x reference)
import functools

import jax
import jax.numpy as jnp
from jax.experimental import pallas as pl
from jax.experimental.pallas import tpu as pltpu

TH = 64
PCAP = 136
NGRP = 8


def _mask_kernel(boxes_ref, boxes_t_ref, pal_ref, et_ref,
                 rmask_ref, cmask_ref, rowchg_ref, palet_ref,
                 *, n_boxes, H, W, vocab):
    h0 = boxes_t_ref[0, 1:2, :].astype(jnp.int32)
    h1 = boxes_t_ref[0, 3:4, :].astype(jnp.int32)
    hh = jax.lax.broadcasted_iota(jnp.int32, (H, n_boxes), 0)
    cover_r = ((hh >= h0) & (hh < h1)).astype(jnp.bfloat16)

    w0 = boxes_ref[0, :, 0:1]
    w1 = boxes_ref[0, :, 2:3]
    ww = jax.lax.broadcasted_iota(jnp.int32, (n_boxes, W), 1)
    cover_c = ((ww >= w0) & (ww < w1)).astype(jnp.bfloat16)

    gi = jax.lax.broadcasted_iota(jnp.int32, (n_boxes, NGRP), 0)
    gj = jax.lax.broadcasted_iota(jnp.int32, (n_boxes, NGRP), 1)
    mbits = jnp.where(gi // 16 == gj,
                      jnp.left_shift(1, gi % 16), 0).astype(jnp.bfloat16)
    rmask = jnp.dot(cover_r, mbits,
                    preferred_element_type=jnp.float32
                    ).astype(jnp.int32)
    rmask_ref[0] = rmask
    cmask_ref[0] = jnp.dot(mbits.T, cover_c,
                           preferred_element_type=jnp.float32
                           ).astype(jnp.int32)

    neq = jnp.max((rmask[1:, :] != rmask[:-1, :]).astype(jnp.int32),
                  axis=1)
    rowchg = jnp.concatenate([jnp.ones((1,), jnp.int32), neq])
    rowchg_ref[0, 0] = rowchg

    pal_row = pal_ref[0, 0, :]
    iota_v = jax.lax.broadcasted_iota(jnp.int32, (vocab, PCAP), 0)
    ohsel = (iota_v == pal_row[None, :]).astype(jnp.bfloat16)
    palet_ref[0] = jnp.dot(et_ref[...], ohsel,
                           preferred_element_type=jnp.float32
                           ).astype(jnp.bfloat16)


def _embed_kernel(rowchg_ref, rmask_ref, cmask_ref, palet_ref, out_ref,
                  cur_ref, *, W, n_bands):
    b = pl.program_id(0)
    band = pl.program_id(1)
    rb = rmask_ref[0]
    cm = cmask_ref[0]
    palet = palet_ref[0]

    idx = jnp.zeros((TH, W), dtype=jnp.int32)
    for j in range(NGRP):
        v = rb[:, j:j + 1] & cm[j, :][None, :]
        p = 31 - jax.lax.clz(v)
        idx = jnp.where(v != 0, 16 * j + p + 1, idx)
    idx_bf = idx.astype(jnp.bfloat16)

    iota_p = jax.lax.broadcasted_iota(jnp.int32, (PCAP, W), 0).astype(
        jnp.bfloat16)
    one = jnp.bfloat16(1.0)
    zero = jnp.bfloat16(0.0)
    for i in range(TH):
        changed = rowchg_ref[b, 0, band * TH + i]

        @pl.when(changed != 0)
        def _():
            oh = jnp.where(iota_p == idx_bf[i, :][None, :], one, zero)
            cur_ref[...] = jnp.dot(palet, oh,
                                   preferred_element_type=jnp.float32)

        out_ref[0, :, i, :] = cur_ref[...]


@jax.jit
def kernel(img, gt_ctexts, gt_cbboxes, embedding_weight):
    B, _, H, W = img.shape
    vocab, D = embedding_weight.shape
    L, Lb = gt_ctexts.shape[1], gt_cbboxes.shape[1]
    T, Tb = gt_ctexts.shape[2], gt_cbboxes.shape[2]
    n_lines, n_tok = min(L, Lb), min(T, Tb)
    N = n_lines * n_tok
    n_bands = H // TH

    toks = gt_ctexts[:, :n_lines, :n_tok].reshape(B, N)
    boxes = jnp.rint(gt_cbboxes[:, :n_lines, :n_tok, :]).astype(
        jnp.int32).reshape(B, N, 4)
    boxes_t = jnp.transpose(boxes, (0, 2, 1))
    pal = jnp.concatenate(
        [jnp.zeros((B, 1), jnp.int32), toks,
         jnp.zeros((B, PCAP - 1 - N), jnp.int32)], axis=1).reshape(B, 1, PCAP)
    et = embedding_weight.T.astype(jnp.bfloat16)

    rmask, cmask, rowchg, palet = pl.pallas_call(
        functools.partial(_mask_kernel, n_boxes=N, H=H, W=W, vocab=vocab),
        grid=(B,),
        in_specs=[
            pl.BlockSpec((1, N, 4), lambda b: (b, 0, 0)),
            pl.BlockSpec((1, 4, N), lambda b: (b, 0, 0)),
            pl.BlockSpec((1, 1, PCAP), lambda b: (b, 0, 0)),
            pl.BlockSpec((D, vocab), lambda b: (0, 0)),
        ],
        out_specs=[
            pl.BlockSpec((1, H, NGRP), lambda b: (b, 0, 0)),
            pl.BlockSpec((1, NGRP, W), lambda b: (b, 0, 0)),
            pl.BlockSpec((1, 1, H), lambda b: (b, 0, 0)),
            pl.BlockSpec((1, D, PCAP), lambda b: (b, 0, 0)),
        ],
        out_shape=[
            jax.ShapeDtypeStruct((B, H, NGRP), jnp.int32),
            jax.ShapeDtypeStruct((B, NGRP, W), jnp.int32),
            jax.ShapeDtypeStruct((B, 1, H), jnp.int32),
            jax.ShapeDtypeStruct((B, D, PCAP), jnp.bfloat16),
        ],
    )(boxes, boxes_t, pal, et)

    out = pl.pallas_call(
        functools.partial(_embed_kernel, W=W, n_bands=n_bands),
        grid_spec=pltpu.PrefetchScalarGridSpec(
            num_scalar_prefetch=1,
            grid=(B, n_bands),
            in_specs=[
                pl.BlockSpec((1, TH, NGRP), lambda b, h, rc: (b, h, 0)),
                pl.BlockSpec((1, NGRP, W), lambda b, h, rc: (b, 0, 0)),
                pl.BlockSpec((1, D, PCAP), lambda b, h, rc: (b, 0, 0)),
            ],
            out_specs=pl.BlockSpec((1, D, TH, W),
                                   lambda b, h, rc: (b, 0, h, 0)),
            scratch_shapes=[pltpu.VMEM((D, W), jnp.float32)],
        ),
        out_shape=jax.ShapeDtypeStruct((B, D, H, W), jnp.float32),
    )(rowchg, rmask, cmask, palet)
    return out


# speedup vs baseline: 1.3386x; 1.3386x over previous
import dataclasses
import functools

import jax
import jax.numpy as jnp
from jax.experimental import pallas as pl
from jax.experimental.pallas import tpu as pltpu
from jax.experimental.pallas import tpu_sc as plsc

TH = 64
PCAP = 136
NGRP = 8
SCL = 16


def _sc_cmask_kernel(boxes_t_hbm, cm_hbm, w0_vmem, w1_vmem, col_vmem,
                     *, B, W):
    c = jax.lax.axis_index("core")
    s = jax.lax.axis_index("subcore")
    tid = c * 16 + s

    @pl.when(tid < B * NGRP)
    def _():
        b = tid // NGRP
        j = tid % NGRP
        pltpu.sync_copy(boxes_t_hbm.at[b, 0, pl.ds(j * SCL, SCL)], w0_vmem)
        pltpu.sync_copy(boxes_t_hbm.at[b, 2, pl.ds(j * SCL, SCL)], w1_vmem)
        lane = jax.lax.broadcasted_iota(jnp.int32, (SCL,), 0)
        w0v = w0_vmem[...]
        w1v = w1_vmem[...]
        zero = jnp.zeros((SCL,), jnp.int32)
        w0s = [jnp.sum(jnp.where(lane == i, w0v, zero)) for i in range(SCL)]
        w1s = [jnp.sum(jnp.where(lane == i, w1v, zero)) for i in range(SCL)]

        @pl.loop(0, W, step=SCL)
        def _(c0):
            ww = lane + c0
            acc = zero
            for i in range(SCL):
                acc = acc | jnp.where((ww >= w0s[i]) & (ww < w1s[i]),
                                      jnp.int32(1 << i), jnp.int32(0))
            col_vmem[pl.ds(c0, SCL)] = acc

        pltpu.sync_copy(col_vmem, cm_hbm.at[b, j, :])


def _cmask_sc(boxes_t, B, W):
    cp = pltpu.CompilerParams()
    if "needs_layout_passes" in pltpu.CompilerParams.__dataclass_fields__:
        cp = dataclasses.replace(cp, needs_layout_passes=False)
    kern = pl.kernel(
        functools.partial(_sc_cmask_kernel, B=B, W=W),
        out_type=jax.ShapeDtypeStruct((B, NGRP, W), jnp.int32),
        mesh=plsc.VectorSubcoreMesh(core_axis_name="core",
                                    subcore_axis_name="subcore"),
        scratch_types=[pltpu.VMEM((SCL,), jnp.int32),
                       pltpu.VMEM((SCL,), jnp.int32),
                       pltpu.VMEM((W,), jnp.int32)],
        compiler_params=cp,
    )
    return kern(boxes_t)


def _mask_kernel(boxes_t_ref, pal_ref, et_ref, rmask_ref, palet_ref,
                 *, n_boxes, H, vocab):
    h0 = boxes_t_ref[0, 1:2, :].astype(jnp.int32)
    h1 = boxes_t_ref[0, 3:4, :].astype(jnp.int32)
    hh = jax.lax.broadcasted_iota(jnp.int32, (H, n_boxes), 0)
    cover_r = ((hh >= h0) & (hh < h1)).astype(jnp.bfloat16)

    gi = jax.lax.broadcasted_iota(jnp.int32, (n_boxes, NGRP), 0)
    gj = jax.lax.broadcasted_iota(jnp.int32, (n_boxes, NGRP), 1)
    mbits = jnp.where(gi // 16 == gj,
                      jnp.left_shift(1, gi % 16), 0).astype(jnp.bfloat16)
    rmask_ref[0] = jnp.dot(cover_r, mbits,
                           preferred_element_type=jnp.float32
                           ).astype(jnp.int32)

    pal_row = pal_ref[0, 0, :]
    iota_v = jax.lax.broadcasted_iota(jnp.int32, (vocab, PCAP), 0)
    ohsel = (iota_v == pal_row[None, :]).astype(jnp.bfloat16)
    palet_ref[0] = jnp.dot(et_ref[...], ohsel,
                           preferred_element_type=jnp.float32
                           ).astype(jnp.bfloat16)


def _embed_kernel(rmask_ref, cmask_ref, palet_ref, out_ref, *, W):
    rb = rmask_ref[0]
    cm = cmask_ref[0]
    palet = palet_ref[0]

    idx = jnp.zeros((TH, W), dtype=jnp.int32)
    for j in range(NGRP):
        v = rb[:, j:j + 1] & cm[j, :][None, :]
        p = 31 - jax.lax.clz(v)
        idx = jnp.where(v != 0, 16 * j + p + 1, idx)
    idx_bf = idx.astype(jnp.bfloat16)

    iota_p = jax.lax.broadcasted_iota(jnp.int32, (PCAP, W), 0).astype(
        jnp.bfloat16)
    one = jnp.bfloat16(1.0)
    zero = jnp.bfloat16(0.0)
    for i in range(TH):
        oh = jnp.where(iota_p == idx_bf[i, :][None, :], one, zero)
        out_ref[0, :, i, :] = jnp.dot(palet, oh,
                                      preferred_element_type=jnp.float32)


@jax.jit
def kernel(img, gt_ctexts, gt_cbboxes, embedding_weight):
    B, _, H, W = img.shape
    vocab, D = embedding_weight.shape
    L, Lb = gt_ctexts.shape[1], gt_cbboxes.shape[1]
    T, Tb = gt_ctexts.shape[2], gt_cbboxes.shape[2]
    n_lines, n_tok = min(L, Lb), min(T, Tb)
    N = n_lines * n_tok
    n_bands = H // TH

    toks = gt_ctexts[:, :n_lines, :n_tok].reshape(B, N)
    boxes = jnp.rint(gt_cbboxes[:, :n_lines, :n_tok, :]).astype(
        jnp.int32).reshape(B, N, 4)
    boxes_t = jnp.transpose(boxes, (0, 2, 1))
    pal = jnp.concatenate(
        [jnp.zeros((B, 1), jnp.int32), toks,
         jnp.zeros((B, PCAP - 1 - N), jnp.int32)], axis=1).reshape(B, 1, PCAP)
    et = embedding_weight.T.astype(jnp.bfloat16)

    cmask = _cmask_sc(boxes_t, B, W)

    rmask, palet = pl.pallas_call(
        functools.partial(_mask_kernel, n_boxes=N, H=H, vocab=vocab),
        grid=(B,),
        in_specs=[
            pl.BlockSpec((1, 4, N), lambda b: (b, 0, 0)),
            pl.BlockSpec((1, 1, PCAP), lambda b: (b, 0, 0)),
            pl.BlockSpec((D, vocab), lambda b: (0, 0)),
        ],
        out_specs=[
            pl.BlockSpec((1, H, NGRP), lambda b: (b, 0, 0)),
            pl.BlockSpec((1, D, PCAP), lambda b: (b, 0, 0)),
        ],
        out_shape=[
            jax.ShapeDtypeStruct((B, H, NGRP), jnp.int32),
            jax.ShapeDtypeStruct((B, D, PCAP), jnp.bfloat16),
        ],
    )(boxes_t, pal, et)

    out = pl.pallas_call(
        functools.partial(_embed_kernel, W=W),
        grid=(B, n_bands),
        in_specs=[
            pl.BlockSpec((1, TH, NGRP), lambda b, h: (b, h, 0)),
            pl.BlockSpec((1, NGRP, W), lambda b, h: (b, 0, 0)),
            pl.BlockSpec((1, D, PCAP), lambda b, h: (b, 0, 0)),
        ],
        out_specs=pl.BlockSpec((1, D, TH, W), lambda b, h: (b, 0, h, 0)),
        out_shape=jax.ShapeDtypeStruct((B, D, H, W), jnp.float32),
    )(rmask, cmask, palet)
    return out


# speedup vs baseline: 1.5657x; 1.1696x over previous
import functools

import jax
import jax.numpy as jnp
from jax.experimental import pallas as pl

TH = 128
PCAP = 136
NGRP = 8


def _mask_kernel(boxes_ref, boxes_t_ref, pal_ref, et_ref,
                 rmask_ref, cmask_ref, palet_ref, *, n_boxes, H, W, vocab):
    h0 = boxes_t_ref[0, 1:2, :].astype(jnp.int32)
    h1 = boxes_t_ref[0, 3:4, :].astype(jnp.int32)
    hh = jax.lax.broadcasted_iota(jnp.int32, (H, n_boxes), 0)
    cover_r = ((hh >= h0) & (hh < h1)).astype(jnp.bfloat16)

    w0 = boxes_ref[0, :, 0:1]
    w1 = boxes_ref[0, :, 2:3]
    ww = jax.lax.broadcasted_iota(jnp.int32, (n_boxes, W), 1)
    cover_c = ((ww >= w0) & (ww < w1)).astype(jnp.bfloat16)

    gi = jax.lax.broadcasted_iota(jnp.int32, (n_boxes, NGRP), 0)
    gj = jax.lax.broadcasted_iota(jnp.int32, (n_boxes, NGRP), 1)
    mbits = jnp.where(gi // 16 == gj,
                      jnp.left_shift(1, gi % 16), 0).astype(jnp.bfloat16)
    rmask_ref[0] = jnp.dot(cover_r, mbits,
                           preferred_element_type=jnp.float32
                           ).astype(jnp.int32)
    cmask_ref[0] = jnp.dot(mbits.T, cover_c,
                           preferred_element_type=jnp.float32
                           ).astype(jnp.int32)

    pal_row = pal_ref[0, 0, :]
    iota_v = jax.lax.broadcasted_iota(jnp.int32, (vocab, PCAP), 0)
    ohsel = (iota_v == pal_row[None, :]).astype(jnp.bfloat16)
    palet_ref[0] = jnp.dot(et_ref[...], ohsel,
                           preferred_element_type=jnp.float32
                           ).astype(jnp.bfloat16)


def _embed_kernel(rmask_ref, cmask_ref, palet_ref, out_ref, *, W):
    rb = rmask_ref[0]
    cm = cmask_ref[0]
    palet = palet_ref[0]

    idx = jnp.zeros((TH, W), dtype=jnp.int32)
    for j in range(NGRP):
        v = rb[:, j:j + 1] & cm[j, :][None, :]
        p = 31 - jax.lax.clz(v)
        idx = jnp.where(v != 0, 16 * j + p + 1, idx)
    idx_bf = idx.astype(jnp.bfloat16)

    iota_p = jax.lax.broadcasted_iota(jnp.int32, (PCAP, W), 0).astype(
        jnp.bfloat16)
    one = jnp.bfloat16(1.0)
    zero = jnp.bfloat16(0.0)
    for i in range(TH):
        oh = jnp.where(iota_p == idx_bf[i, :][None, :], one, zero)
        out_ref[0, :, i, :] = jnp.dot(palet, oh,
                                      preferred_element_type=jnp.float32)


@jax.jit
def kernel(img, gt_ctexts, gt_cbboxes, embedding_weight):
    B, _, H, W = img.shape
    vocab, D = embedding_weight.shape
    L, Lb = gt_ctexts.shape[1], gt_cbboxes.shape[1]
    T, Tb = gt_ctexts.shape[2], gt_cbboxes.shape[2]
    n_lines, n_tok = min(L, Lb), min(T, Tb)
    N = n_lines * n_tok
    n_bands = H // TH

    toks = gt_ctexts[:, :n_lines, :n_tok].reshape(B, N)
    boxes = jnp.rint(gt_cbboxes[:, :n_lines, :n_tok, :]).astype(
        jnp.int32).reshape(B, N, 4)
    boxes_t = jnp.transpose(boxes, (0, 2, 1))
    pal = jnp.concatenate(
        [jnp.zeros((B, 1), jnp.int32), toks,
         jnp.zeros((B, PCAP - 1 - N), jnp.int32)], axis=1).reshape(B, 1, PCAP)
    et = embedding_weight.T.astype(jnp.bfloat16)

    rmask, cmask, palet = pl.pallas_call(
        functools.partial(_mask_kernel, n_boxes=N, H=H, W=W, vocab=vocab),
        grid=(B,),
        in_specs=[
            pl.BlockSpec((1, N, 4), lambda b: (b, 0, 0)),
            pl.BlockSpec((1, 4, N), lambda b: (b, 0, 0)),
            pl.BlockSpec((1, 1, PCAP), lambda b: (b, 0, 0)),
            pl.BlockSpec((D, vocab), lambda b: (0, 0)),
        ],
        out_specs=[
            pl.BlockSpec((1, H, NGRP), lambda b: (b, 0, 0)),
            pl.BlockSpec((1, NGRP, W), lambda b: (b, 0, 0)),
            pl.BlockSpec((1, D, PCAP), lambda b: (b, 0, 0)),
        ],
        out_shape=[
            jax.ShapeDtypeStruct((B, H, NGRP), jnp.int32),
            jax.ShapeDtypeStruct((B, NGRP, W), jnp.int32),
            jax.ShapeDtypeStruct((B, D, PCAP), jnp.bfloat16),
        ],
    )(boxes, boxes_t, pal, et)

    out = pl.pallas_call(
        functools.partial(_embed_kernel, W=W),
        grid=(B, n_bands),
        in_specs=[
            pl.BlockSpec((1, TH, NGRP), lambda b, h: (b, h, 0)),
            pl.BlockSpec((1, NGRP, W), lambda b, h: (b, 0, 0)),
            pl.BlockSpec((1, D, PCAP), lambda b, h: (b, 0, 0)),
        ],
        out_specs=pl.BlockSpec((1, D, TH, W), lambda b, h: (b, 0, h, 0)),
        out_shape=jax.ShapeDtypeStruct((B, D, H, W), jnp.float32),
    )(rmask, cmask, palet)
    return out


# speedup vs baseline: 1.5967x; 1.0198x over previous
import functools

import jax
import jax.numpy as jnp
from jax.experimental import pallas as pl
from jax.experimental.pallas import tpu as pltpu

TH = 64
PCAP = 136
NGRP = 8


def _mask_kernel(boxes_ref, boxes_t_ref, pal_ref, et_ref,
                 rmask_ref, cmask_ref, palet_ref, *, n_boxes, H, W, vocab):
    h0 = boxes_t_ref[0, 1:2, :].astype(jnp.int32)
    h1 = boxes_t_ref[0, 3:4, :].astype(jnp.int32)
    hh = jax.lax.broadcasted_iota(jnp.int32, (H, n_boxes), 0)
    cover_r = ((hh >= h0) & (hh < h1)).astype(jnp.bfloat16)

    w0 = boxes_ref[0, :, 0:1]
    w1 = boxes_ref[0, :, 2:3]
    ww = jax.lax.broadcasted_iota(jnp.int32, (n_boxes, W), 1)
    cover_c = ((ww >= w0) & (ww < w1)).astype(jnp.bfloat16)

    gi = jax.lax.broadcasted_iota(jnp.int32, (n_boxes, NGRP), 0)
    gj = jax.lax.broadcasted_iota(jnp.int32, (n_boxes, NGRP), 1)
    mbits = jnp.where(gi // 16 == gj,
                      jnp.left_shift(1, gi % 16), 0).astype(jnp.bfloat16)
    rmask_ref[0] = jnp.dot(cover_r, mbits,
                           preferred_element_type=jnp.float32
                           ).astype(jnp.int32)
    cmask_ref[0] = jnp.dot(mbits.T, cover_c,
                           preferred_element_type=jnp.float32
                           ).astype(jnp.int32)

    pal_row = pal_ref[0, 0, :]
    iota_v = jax.lax.broadcasted_iota(jnp.int32, (vocab, PCAP), 0)
    ohsel = (iota_v == pal_row[None, :]).astype(jnp.bfloat16)
    palet_ref[0] = jnp.dot(et_ref[...], ohsel,
                           preferred_element_type=jnp.float32
                           ).astype(jnp.bfloat16)


def _embed_kernel(rmask_ref, cmask_ref, palet_ref, out_ref, *, W):
    rb = rmask_ref[0]
    cm = cmask_ref[0]
    palet = palet_ref[0]

    idx = jnp.zeros((TH, W), dtype=jnp.int32)
    for j in range(NGRP):
        v = rb[:, j:j + 1] & cm[j, :][None, :]
        p = 31 - jax.lax.clz(v)
        idx = jnp.where(v != 0, 16 * j + p + 1, idx)
    idx_bf = idx.astype(jnp.bfloat16)

    iota_p = jax.lax.broadcasted_iota(jnp.int32, (PCAP, W), 0).astype(
        jnp.bfloat16)
    one = jnp.bfloat16(1.0)
    zero = jnp.bfloat16(0.0)
    for i in range(TH):
        oh = jnp.where(iota_p == idx_bf[i, :][None, :], one, zero)
        out_ref[0, :, i, :] = jnp.dot(palet, oh,
                                      preferred_element_type=jnp.float32)


@jax.jit
def kernel(img, gt_ctexts, gt_cbboxes, embedding_weight):
    B, _, H, W = img.shape
    vocab, D = embedding_weight.shape
    L, Lb = gt_ctexts.shape[1], gt_cbboxes.shape[1]
    T, Tb = gt_ctexts.shape[2], gt_cbboxes.shape[2]
    n_lines, n_tok = min(L, Lb), min(T, Tb)
    N = n_lines * n_tok
    n_bands = H // TH

    toks = gt_ctexts[:, :n_lines, :n_tok].reshape(B, N)
    boxes = jnp.rint(gt_cbboxes[:, :n_lines, :n_tok, :]).astype(
        jnp.int32).reshape(B, N, 4)
    boxes_t = jnp.transpose(boxes, (0, 2, 1))
    pal = jnp.concatenate(
        [jnp.zeros((B, 1), jnp.int32), toks,
         jnp.zeros((B, PCAP - 1 - N), jnp.int32)], axis=1).reshape(B, 1, PCAP)
    et = embedding_weight.T.astype(jnp.bfloat16)

    rmask, cmask, palet = pl.pallas_call(
        functools.partial(_mask_kernel, n_boxes=N, H=H, W=W, vocab=vocab),
        grid=(B,),
        in_specs=[
            pl.BlockSpec((1, N, 4), lambda b: (b, 0, 0)),
            pl.BlockSpec((1, 4, N), lambda b: (b, 0, 0)),
            pl.BlockSpec((1, 1, PCAP), lambda b: (b, 0, 0)),
            pl.BlockSpec((D, vocab), lambda b: (0, 0)),
        ],
        out_specs=[
            pl.BlockSpec((1, H, NGRP), lambda b: (b, 0, 0)),
            pl.BlockSpec((1, NGRP, W), lambda b: (b, 0, 0)),
            pl.BlockSpec((1, D, PCAP), lambda b: (b, 0, 0)),
        ],
        out_shape=[
            jax.ShapeDtypeStruct((B, H, NGRP), jnp.int32),
            jax.ShapeDtypeStruct((B, NGRP, W), jnp.int32),
            jax.ShapeDtypeStruct((B, D, PCAP), jnp.bfloat16),
        ],
    )(boxes, boxes_t, pal, et)

    out = pl.pallas_call(
        functools.partial(_embed_kernel, W=W),
        grid=(B, n_bands),
        in_specs=[
            pl.BlockSpec((1, TH, NGRP), lambda b, h: (b, h, 0)),
            pl.BlockSpec((1, NGRP, W), lambda b, h: (b, 0, 0)),
            pl.BlockSpec((1, D, PCAP), lambda b, h: (b, 0, 0)),
        ],
        out_specs=pl.BlockSpec((1, D, TH, W), lambda b, h: (b, 0, h, 0)),
        out_shape=jax.ShapeDtypeStruct((B, D, H, W), jnp.float32),
        compiler_params=pltpu.CompilerParams(
            dimension_semantics=("parallel", "parallel")),
    )(rmask, cmask, palet)
    return out


# speedup vs baseline: 1.6318x; 1.0220x over previous
import functools

import jax
import jax.numpy as jnp
from jax.experimental import pallas as pl
from jax.experimental.pallas import tpu as pltpu

TH = 64
PCAP = 136
NGRP = 8


def _embed_kernel(boxes_ref, boxes_t_ref, pal_ref, et_ref, out_ref,
                  rmask_scr, cmask_scr, palet_scr, *, n_boxes, H, W, vocab):
    band = pl.program_id(1)

    @pl.when(band == 0)
    def _():
        h0 = boxes_t_ref[0, 1:2, :]
        h1 = boxes_t_ref[0, 3:4, :]
        hh = jax.lax.broadcasted_iota(jnp.int32, (H, n_boxes), 0)
        cover_r = ((hh >= h0) & (hh < h1)).astype(jnp.bfloat16)

        w0 = boxes_ref[0, :, 0:1]
        w1 = boxes_ref[0, :, 2:3]
        ww = jax.lax.broadcasted_iota(jnp.int32, (n_boxes, W), 1)
        cover_c = ((ww >= w0) & (ww < w1)).astype(jnp.bfloat16)

        gi = jax.lax.broadcasted_iota(jnp.int32, (n_boxes, NGRP), 0)
        gj = jax.lax.broadcasted_iota(jnp.int32, (n_boxes, NGRP), 1)
        mbits = jnp.where(gi // 16 == gj,
                          jnp.left_shift(1, gi % 16), 0).astype(jnp.bfloat16)
        rmask_scr[...] = jnp.dot(cover_r, mbits,
                                 preferred_element_type=jnp.float32
                                 ).astype(jnp.int32)
        cmask_scr[...] = jnp.dot(mbits.T, cover_c,
                                 preferred_element_type=jnp.float32
                                 ).astype(jnp.int32)

        pal_row = pal_ref[0, 0, :]
        iota_v = jax.lax.broadcasted_iota(jnp.int32, (vocab, PCAP), 0)
        ohsel = (iota_v == pal_row[None, :]).astype(jnp.bfloat16)
        palet_scr[...] = jnp.dot(et_ref[...], ohsel,
                                 preferred_element_type=jnp.float32
                                 ).astype(jnp.bfloat16)

    rb = rmask_scr[pl.ds(band * TH, TH), :]
    cm = cmask_scr[...]
    palet = palet_scr[...]

    idx = jnp.zeros((TH, W), dtype=jnp.int32)
    for j in range(NGRP):
        v = rb[:, j:j + 1] & cm[j, :][None, :]
        p = 31 - jax.lax.clz(v)
        idx = jnp.where(v != 0, 16 * j + p + 1, idx)
    idx_bf = idx.astype(jnp.bfloat16)

    iota_p = jax.lax.broadcasted_iota(jnp.int32, (PCAP, W), 0).astype(
        jnp.bfloat16)
    one = jnp.bfloat16(1.0)
    zero = jnp.bfloat16(0.0)
    for i in range(TH):
        oh = jnp.where(iota_p == idx_bf[i, :][None, :], one, zero)
        out_ref[0, :, i, :] = jnp.dot(palet, oh,
                                      preferred_element_type=jnp.float32)


@jax.jit
def kernel(img, gt_ctexts, gt_cbboxes, embedding_weight):
    B, _, H, W = img.shape
    vocab, D = embedding_weight.shape
    L, Lb = gt_ctexts.shape[1], gt_cbboxes.shape[1]
    T, Tb = gt_ctexts.shape[2], gt_cbboxes.shape[2]
    n_lines, n_tok = min(L, Lb), min(T, Tb)
    N = n_lines * n_tok
    n_bands = H // TH

    toks = gt_ctexts[:, :n_lines, :n_tok].reshape(B, N)
    boxes = jnp.rint(gt_cbboxes[:, :n_lines, :n_tok, :]).astype(
        jnp.int32).reshape(B, N, 4)
    boxes_t = jnp.transpose(boxes, (0, 2, 1))
    pal = jnp.concatenate(
        [jnp.zeros((B, 1), jnp.int32), toks,
         jnp.zeros((B, PCAP - 1 - N), jnp.int32)], axis=1).reshape(B, 1, PCAP)
    et = embedding_weight.T.astype(jnp.bfloat16)

    out = pl.pallas_call(
        functools.partial(_embed_kernel, n_boxes=N, H=H, W=W, vocab=vocab),
        grid=(B, n_bands),
        in_specs=[
            pl.BlockSpec((1, N, 4), lambda b, h: (b, 0, 0)),
            pl.BlockSpec((1, 4, N), lambda b, h: (b, 0, 0)),
            pl.BlockSpec((1, 1, PCAP), lambda b, h: (b, 0, 0)),
            pl.BlockSpec((D, vocab), lambda b, h: (0, 0)),
        ],
        out_specs=pl.BlockSpec((1, D, TH, W), lambda b, h: (b, 0, h, 0)),
        out_shape=jax.ShapeDtypeStruct((B, D, H, W), jnp.float32),
        scratch_shapes=[
            pltpu.VMEM((H, NGRP), jnp.int32),
            pltpu.VMEM((NGRP, W), jnp.int32),
            pltpu.VMEM((D, PCAP), jnp.bfloat16),
        ],
    )(boxes, boxes_t, pal, et)
    return out
